# baseline (speedup 1.0000x reference)
import functools

import numpy as np
import jax
import jax.numpy as jnp
from jax import lax
from jax.experimental import pallas as pl
from jax.experimental.pallas import tpu as pltpu
from jax.experimental.pallas import tpu_sc as plsc

_NUM_LEVELS = 20
_LOG2_HASHMAP = 21
_BASE_RES = 16
_DESIRED_RES = 8193
_N = 262144
_P1 = np.uint32(2654435761).astype(np.int32)
_P2 = np.int32(805459861)
_MASK = np.int32(2**_LOG2_HASHMAP - 1)
_NDENSE = 7

_LANES = 16
_NC = 2
_NS = 16
_NW = _NC * _NS
_PW = _N // _NW
_C = 512
_NCH = _PW // _C


def _levels_meta():
    scale = np.exp(np.log(_DESIRED_RES / _BASE_RES) / (_NUM_LEVELS - 1))
    maxp = 2 ** _LOG2_HASHMAP
    res, sizes = [], []
    for i in range(_NUM_LEVELS):
        r = int(np.ceil(_BASE_RES * scale ** i))
        res.append(r)
        n = min(maxp, (r + 1) ** 3)
        sizes.append(int(np.ceil(n / 8.0) * 8))
    offs = [0]
    for s in sizes:
        offs.append(offs[-1] + s)
    return res, offs


_RES, _OFFS = _levels_meta()
_T = _OFFS[-1]
_TDIV8 = _T // 8


def _splat(v, dt):
    return np.full((_LANES,), v, dt)


_PVF = np.stack([_splat(float(r), np.float32) for r in _RES])
_PVI = np.stack([
    np.stack([
        _splat(r - 1, np.int32),
        _splat(r + 1, np.int32),
        _splat((r + 1) * (r + 1), np.int32),
        _splat(_OFFS[l], np.int32),
    ])
    for l, r in enumerate(_RES)
])


def _enc_body(px_h, py_h, pz_h, tab_h, pvf_h, pvi_h, enc_h,
              px, py, pz, pvf, pvi,
              ib0, ib1, ib2, ib3, ib4, ib5, ib6, ib7,
              jb0, jb1, jb2, jb3, jb4, jb5, jb6, jb7,
              imod, wb,
              ra0, ra1, ra2, ra3, ra4, ra5, ra6, ra7,
              rb0, rb1, rb2, rb3, rb4, rb5, rb6, rb7,
              encb, sem):
    ibs = [ib0, ib1, ib2, ib3, ib4, ib5, ib6, ib7]
    jbs = [jb0, jb1, jb2, jb3, jb4, jb5, jb6, jb7]
    ras = [ra0, ra1, ra2, ra3, ra4, ra5, ra6, ra7]
    rbs = [rb0, rb1, rb2, rb3, rb4, rb5, rb6, rb7]

    wid = lax.axis_index("s") * _NC + lax.axis_index("c")
    pltpu.sync_copy(pvf_h, pvf)
    pltpu.sync_copy(pvi_h, pvi)
    iota = lax.iota(jnp.int32, _LANES)
    base0 = wid * _PW

    def chunk_body(ch, carry):
        base = base0 + ch * _C
        pltpu.sync_copy(px_h.at[pl.ds(base, _C)], px)
        pltpu.sync_copy(py_h.at[pl.ds(base, _C)], py)
        pltpu.sync_copy(pz_h.at[pl.ds(base, _C)], pz)

        def do_level(l, hashed):
            res_v = pvf[l]
            resm1 = pvi[l, 0]
            s1 = pvi[l, 1]
            s2 = pvi[l, 2]
            off = pvi[l, 3]

            def idxw_body(i, c2):
                o = i * _LANES
                xs = px[pl.ds(o, _LANES)] * res_v
                ys = py[pl.ds(o, _LANES)] * res_v
                zs = pz[pl.ds(o, _LANES)] * res_v
                xi = jnp.minimum(xs.astype(jnp.int32), resm1)
                yi = jnp.minimum(ys.astype(jnp.int32), resm1)
                zi = jnp.minimum(zs.astype(jnp.int32), resm1)
                fx = xs - xi.astype(jnp.float32)
                fy = ys - yi.astype(jnp.float32)
                fz = zs - zi.astype(jnp.float32)
                if hashed:
                    hx = (xi, xi + 1)
                    hy0 = yi * _P1
                    hz0 = zi * _P2
                    hy = (hy0, hy0 + _P1)
                    hz = (hz0, hz0 + _P2)
                    idxs = [((hx[c & 1] ^ hy[(c >> 1) & 1] ^ hz[(c >> 2) & 1])
                             & _MASK) + off
                            for c in range(8)]
                else:
                    xc = (xi, xi + 1)
                    ys0 = yi * s1
                    zs0 = zi * s2
                    yc = (ys0, ys0 + s1)
                    zc = (zs0, zs0 + s2)
                    idxs = [xc[c & 1] + yc[(c >> 1) & 1] + zc[(c >> 2) & 1]
                            + off
                            for c in range(8)]
                for c in range(8):
                    blk = lax.shift_right_logical(idxs[c], 3)
                    ibs[c][pl.ds(o, _LANES)] = blk
                    jbs[c][pl.ds(o, _LANES)] = blk + _TDIV8
                    imod[c, pl.ds(o, _LANES)] = idxs[c] & 7
                wx = (1.0 - fx, fx)
                wy = (1.0 - fy, fy)
                wz = (1.0 - fz, fz)
                wxy = [wx[b0] * wy[b1] for b1 in range(2) for b0 in range(2)]
                for c in range(8):
                    wb[c, pl.ds(o, _LANES)] = (wxy[((c >> 1) & 1) * 2 + (c & 1)]
                                               * wz[(c >> 2) & 1])
                return c2

            lax.fori_loop(0, _C // _LANES, idxw_body, 0)

            descs = ([pltpu.async_copy(tab_h.at[ibs[c]], ras[c], sem)
                      for c in range(8)]
                     + [pltpu.async_copy(tab_h.at[jbs[c]], rbs[c], sem)
                        for c in range(8)])
            for d in descs:
                d.wait()

            def acc_body(i, c2):
                o = i * _LANES
                ipt = iota + jnp.full((_LANES,), o, jnp.int32)
                acc0 = None
                acc1 = None
                for c in range(8):
                    mv = imod[c, pl.ds(o, _LANES)]
                    wv = wb[c, pl.ds(o, _LANES)]
                    r0 = plsc.load_gather(ras[c], [ipt, mv])
                    r1 = plsc.load_gather(rbs[c], [ipt, mv])
                    t0 = wv * r0
                    t1 = wv * r1
                    acc0 = t0 if acc0 is None else acc0 + t0
                    acc1 = t1 if acc1 is None else acc1 + t1
                encb[2 * l, pl.ds(o, _LANES)] = acc0
                encb[2 * l + 1, pl.ds(o, _LANES)] = acc1
                return c2

            lax.fori_loop(0, _C // _LANES, acc_body, 0)
            return 0

        lax.fori_loop(0, _NDENSE, lambda l, c2: do_level(l, False), 0)
        lax.fori_loop(_NDENSE, _NUM_LEVELS, lambda l, c2: do_level(l, True), 0)

        pltpu.sync_copy(encb, enc_h.at[:, pl.ds(base, _C)])
        return carry

    lax.fori_loop(0, _NCH, chunk_body, 0)


_enc_kernel = functools.partial(
    pl.kernel,
    out_type=jax.ShapeDtypeStruct((40, _N), jnp.float32),
    mesh=plsc.VectorSubcoreMesh(core_axis_name="c", subcore_axis_name="s",
                                num_cores=_NC, num_subcores=_NS),
    compiler_params=pltpu.CompilerParams(use_tc_tiling_on_sc=False,
                                         needs_layout_passes=False),
    scratch_types=(
        [pltpu.VMEM((_C,), jnp.float32)] * 3
        + [pltpu.VMEM((_NUM_LEVELS, _LANES), jnp.float32),
           pltpu.VMEM((_NUM_LEVELS, 4, _LANES), jnp.int32)]
        + [pltpu.VMEM((_C,), jnp.int32)] * 16
        + [pltpu.VMEM((8, _C), jnp.int32)]
        + [pltpu.VMEM((8, _C), jnp.float32)]
        + [pltpu.VMEM((_C, 8), jnp.float32)] * 16
        + [pltpu.VMEM((40, _C), jnp.float32),
           pltpu.SemaphoreType.DMA]
    ),
)(_enc_body)


_BT = 2048


def _mlp_body(e_ref, w1_ref, wo_ref, o_ref):
    x = jnp.maximum(
        jnp.dot(w1_ref[...], e_ref[...], preferred_element_type=jnp.float32),
        0.0)
    o_ref[...] = jnp.dot(wo_ref[...], x, preferred_element_type=jnp.float32)


def _mlp(enc, w1, wo):
    return pl.pallas_call(
        _mlp_body,
        grid=(_N // _BT,),
        in_specs=[
            pl.BlockSpec((40, _BT), lambda i: (0, i)),
            pl.BlockSpec((64, 40), lambda i: (0, 0)),
            pl.BlockSpec((8, 64), lambda i: (0, 0)),
        ],
        out_specs=pl.BlockSpec((8, _BT), lambda i: (0, i)),
        out_shape=jax.ShapeDtypeStruct((8, _N), jnp.float32),
    )(enc, w1, wo)


def kernel(positions, bound, table, W1, Wd, Wf):
    bf = jnp.float32(bound)
    pos01 = jnp.clip((positions + bf) / (2.0 * bf), 0.0, 1.0)
    px = pos01[:, 0]
    py = pos01[:, 1]
    pz = pos01[:, 2]
    tcat = jnp.concatenate([table[:, 0], table[:, 1]]).reshape(_T // 4, 8)
    enc = _enc_kernel(px, py, pz, tcat,
                      jnp.asarray(_PVF), jnp.asarray(_PVI))
    wo = jnp.concatenate([Wf, Wd], axis=0)
    out8 = _mlp(enc, W1, wo)
    return (out8[:7, :].T, out8[7:8, :].T)


# speedup vs baseline: 3.0042x; 3.0042x over previous
import functools

import numpy as np
import jax
import jax.numpy as jnp
from jax import lax
from jax.experimental import pallas as pl
from jax.experimental.pallas import tpu as pltpu
from jax.experimental.pallas import tpu_sc as plsc

_NUM_LEVELS = 20
_LOG2_HASHMAP = 21
_BASE_RES = 16
_DESIRED_RES = 8193
_N = 262144
_P1 = np.uint32(2654435761).astype(np.int32)
_P2 = np.int32(805459861)
_MASK = np.int32(2**_LOG2_HASHMAP - 1)
_NDENSE = 7

_LANES = 16
_NC = 2
_NS = 16
_NW = _NC * _NS
_PW = _N // _NW
_C = 512
_NCH = _PW // _C


def _levels_meta():
    scale = np.exp(np.log(_DESIRED_RES / _BASE_RES) / (_NUM_LEVELS - 1))
    maxp = 2 ** _LOG2_HASHMAP
    res, sizes = [], []
    for i in range(_NUM_LEVELS):
        r = int(np.ceil(_BASE_RES * scale ** i))
        res.append(r)
        n = min(maxp, (r + 1) ** 3)
        sizes.append(int(np.ceil(n / 8.0) * 8))
    offs = [0]
    for s in sizes:
        offs.append(offs[-1] + s)
    return res, offs


_RES, _OFFS = _levels_meta()
_T = _OFFS[-1]
_TPAD = (-_T) % 128
_TP = _T + _TPAD


def _splat(v, dt):
    return np.full((_LANES,), v, dt)


_PVF = np.stack([_splat(float(r), np.float32) for r in _RES])
_PVI = np.stack([
    np.stack([
        _splat(r - 1, np.int32),
        _splat(r + 1, np.int32),
        _splat((r + 1) * (r + 1), np.int32),
        _splat(_OFFS[l], np.int32),
    ])
    for l, r in enumerate(_RES)
])


def _enc_body(px_h, py_h, pz_h, t0_h, t1_h, pvf_h, pvi_h, enc_h,
              px, py, pz, pvf, pvi,
              ib0, ib1, ib2, ib3, ib4, ib5, ib6, ib7,
              imod, wb,
              ra0, ra1, ra2, ra3, ra4, ra5, ra6, ra7,
              rb0, rb1, rb2, rb3, rb4, rb5, rb6, rb7,
              encb, sem):
    ibs = [ib0, ib1, ib2, ib3, ib4, ib5, ib6, ib7]
    ras = [ra0, ra1, ra2, ra3, ra4, ra5, ra6, ra7]
    rbs = [rb0, rb1, rb2, rb3, rb4, rb5, rb6, rb7]

    wid = lax.axis_index("s") * _NC + lax.axis_index("c")
    pltpu.sync_copy(pvf_h, pvf)
    pltpu.sync_copy(pvi_h, pvi)
    iota = lax.iota(jnp.int32, _LANES)
    base0 = wid * _PW

    def chunk_body(ch, carry):
        base = base0 + ch * _C
        pltpu.sync_copy(px_h.at[pl.ds(base, _C)], px)
        pltpu.sync_copy(py_h.at[pl.ds(base, _C)], py)
        pltpu.sync_copy(pz_h.at[pl.ds(base, _C)], pz)

        def do_level(l, hashed):
            res_v = pvf[l]
            resm1 = pvi[l, 0]
            s1 = pvi[l, 1]
            s2 = pvi[l, 2]
            off = pvi[l, 3]

            def idxw_body(i, c2):
                o = i * _LANES
                xs = px[pl.ds(o, _LANES)] * res_v
                ys = py[pl.ds(o, _LANES)] * res_v
                zs = pz[pl.ds(o, _LANES)] * res_v
                xi = jnp.minimum(xs.astype(jnp.int32), resm1)
                yi = jnp.minimum(ys.astype(jnp.int32), resm1)
                zi = jnp.minimum(zs.astype(jnp.int32), resm1)
                fx = xs - xi.astype(jnp.float32)
                fy = ys - yi.astype(jnp.float32)
                fz = zs - zi.astype(jnp.float32)
                if hashed:
                    hx = (xi, xi + 1)
                    hy0 = yi * _P1
                    hz0 = zi * _P2
                    hy = (hy0, hy0 + _P1)
                    hz = (hz0, hz0 + _P2)
                    idxs = [((hx[c & 1] ^ hy[(c >> 1) & 1] ^ hz[(c >> 2) & 1])
                             & _MASK) + off
                            for c in range(8)]
                else:
                    xc = (xi, xi + 1)
                    ys0 = yi * s1
                    zs0 = zi * s2
                    yc = (ys0, ys0 + s1)
                    zc = (zs0, zs0 + s2)
                    idxs = [xc[c & 1] + yc[(c >> 1) & 1] + zc[(c >> 2) & 1]
                            + off
                            for c in range(8)]
                for c in range(8):
                    ibs[c][pl.ds(o, _LANES)] = lax.shift_right_logical(
                        idxs[c], 3)
                    imod[c, pl.ds(o, _LANES)] = idxs[c] & 7
                wx = (1.0 - fx, fx)
                wy = (1.0 - fy, fy)
                wz = (1.0 - fz, fz)
                wxy = [wx[b0] * wy[b1] for b1 in range(2) for b0 in range(2)]
                for c in range(8):
                    wb[c, pl.ds(o, _LANES)] = (wxy[((c >> 1) & 1) * 2 + (c & 1)]
                                               * wz[(c >> 2) & 1])
                return c2

            lax.fori_loop(0, _C // _LANES, idxw_body, 0)

            descs = ([pltpu.async_copy(t0_h.at[ibs[c]], ras[c], sem)
                      for c in range(8)]
                     + [pltpu.async_copy(t1_h.at[ibs[c]], rbs[c], sem)
                        for c in range(8)])
            for d in descs:
                d.wait()

            def acc_body(i, c2):
                o = i * _LANES
                ipt = iota + jnp.full((_LANES,), o, jnp.int32)
                acc0 = None
                acc1 = None
                for c in range(8):
                    mv = imod[c, pl.ds(o, _LANES)]
                    wv = wb[c, pl.ds(o, _LANES)]
                    r0 = plsc.load_gather(ras[c], [ipt, mv])
                    r1 = plsc.load_gather(rbs[c], [ipt, mv])
                    t0 = wv * r0
                    t1 = wv * r1
                    acc0 = t0 if acc0 is None else acc0 + t0
                    acc1 = t1 if acc1 is None else acc1 + t1
                encb[2 * l, pl.ds(o, _LANES)] = acc0
                encb[2 * l + 1, pl.ds(o, _LANES)] = acc1
                return c2

            lax.fori_loop(0, _C // _LANES, acc_body, 0)
            return 0

        lax.fori_loop(0, _NDENSE, lambda l, c2: do_level(l, False), 0)
        lax.fori_loop(_NDENSE, _NUM_LEVELS, lambda l, c2: do_level(l, True), 0)

        pltpu.sync_copy(encb, enc_h.at[:, pl.ds(base, _C)])
        return carry

    lax.fori_loop(0, _NCH, chunk_body, 0)


_enc_kernel = functools.partial(
    pl.kernel,
    out_type=jax.ShapeDtypeStruct((40, _N), jnp.float32),
    mesh=plsc.VectorSubcoreMesh(core_axis_name="c", subcore_axis_name="s",
                                num_cores=_NC, num_subcores=_NS),
    compiler_params=pltpu.CompilerParams(use_tc_tiling_on_sc=False,
                                         needs_layout_passes=False),
    scratch_types=(
        [pltpu.VMEM((_C,), jnp.float32)] * 3
        + [pltpu.VMEM((_NUM_LEVELS, _LANES), jnp.float32),
           pltpu.VMEM((_NUM_LEVELS, 4, _LANES), jnp.int32)]
        + [pltpu.VMEM((_C,), jnp.int32)] * 8
        + [pltpu.VMEM((8, _C), jnp.int32)]
        + [pltpu.VMEM((8, _C), jnp.float32)]
        + [pltpu.VMEM((_C, 8), jnp.float32)] * 16
        + [pltpu.VMEM((40, _C), jnp.float32),
           pltpu.SemaphoreType.DMA]
    ),
)(_enc_body)


_RB = 131072


def _split_body(t_ref, o0_ref, o1_ref):
    x = t_ref[...]
    o0_ref[...] = x[0, :]
    o1_ref[...] = x[1, :]


def _split_planes(table_t):
    grid = (_T + _RB - 1) // _RB
    return pl.pallas_call(
        _split_body,
        grid=(grid,),
        in_specs=[pl.BlockSpec((2, _RB), lambda i: (0, i))],
        out_specs=[pl.BlockSpec((_RB,), lambda i: (i,)),
                   pl.BlockSpec((_RB,), lambda i: (i,))],
        out_shape=[jax.ShapeDtypeStruct((_T,), jnp.float32),
                   jax.ShapeDtypeStruct((_T,), jnp.float32)],
    )(table_t)


_BT = 2048


def _mlp_body(e_ref, w1_ref, wo_ref, o_ref):
    x = jnp.maximum(
        jnp.dot(w1_ref[...], e_ref[...], preferred_element_type=jnp.float32),
        0.0)
    o_ref[...] = jnp.dot(wo_ref[...], x, preferred_element_type=jnp.float32)


def _mlp(enc, w1, wo):
    return pl.pallas_call(
        _mlp_body,
        grid=(_N // _BT,),
        in_specs=[
            pl.BlockSpec((40, _BT), lambda i: (0, i)),
            pl.BlockSpec((64, 40), lambda i: (0, 0)),
            pl.BlockSpec((8, 64), lambda i: (0, 0)),
        ],
        out_specs=pl.BlockSpec((8, _BT), lambda i: (0, i)),
        out_shape=jax.ShapeDtypeStruct((8, _N), jnp.float32),
    )(enc, w1, wo)


def kernel(positions, bound, table, W1, Wd, Wf):
    bf = jnp.float32(bound)
    pos01 = jnp.clip((positions + bf) / (2.0 * bf), 0.0, 1.0)
    px = pos01[:, 0]
    py = pos01[:, 1]
    pz = pos01[:, 2]
    t0, t1 = _split_planes(table.T)
    enc = _enc_kernel(px, py, pz,
                      t0.reshape(_T // 8, 8), t1.reshape(_T // 8, 8),
                      jnp.asarray(_PVF), jnp.asarray(_PVI))
    wo = jnp.concatenate([Wf, Wd], axis=0)
    out8 = _mlp(enc, W1, wo)
    return (out8[:7, :].T, out8[7:8, :].T)


# speedup vs baseline: 3.5718x; 1.1889x over previous
import functools

import numpy as np
import jax
import jax.numpy as jnp
from jax import lax
from jax.experimental import pallas as pl
from jax.experimental.pallas import tpu as pltpu
from jax.experimental.pallas import tpu_sc as plsc

_NUM_LEVELS = 20
_LOG2_HASHMAP = 21
_BASE_RES = 16
_DESIRED_RES = 8193
_N = 262144
_P1 = np.uint32(2654435761).astype(np.int32)
_P2 = np.int32(805459861)
_MASK = np.int32(2**_LOG2_HASHMAP - 1)
_NDENSE = 7

_LANES = 16
_NC = 2
_NS = 16
_NW = _NC * _NS
_PW = _N // _NW
_C = 256
_NCH = _PW // _C


def _levels_meta():
    scale = np.exp(np.log(_DESIRED_RES / _BASE_RES) / (_NUM_LEVELS - 1))
    maxp = 2 ** _LOG2_HASHMAP
    res, sizes = [], []
    for i in range(_NUM_LEVELS):
        r = int(np.ceil(_BASE_RES * scale ** i))
        res.append(r)
        n = min(maxp, (r + 1) ** 3)
        sizes.append(int(np.ceil(n / 8.0) * 8))
    offs = [0]
    for s in sizes:
        offs.append(offs[-1] + s)
    return res, offs


_RES, _OFFS = _levels_meta()
_T = _OFFS[-1]
_TPAD = (-_T) % 128
_TP = _T + _TPAD


def _enc_body(px_h, py_h, pz_h, t0_h, t1_h, enc_h,
              px, py, pz, ib2, im2, wb2, ra2, rb2, encb, sem0, sem1):
    sems = (sem0, sem1)
    wid = lax.axis_index("s") * _NC + lax.axis_index("c")
    iota = lax.iota(jnp.int32, _LANES)
    base0 = wid * _PW

    def idxw(l, s):
        r = _RES[l]
        off = _OFFS[l]
        hashed = l >= _NDENSE
        r_f = float(r)
        s1 = r + 1
        s2 = s1 * s1

        def body(i, c2):
            o = i * _LANES
            xs = px[pl.ds(o, _LANES)] * r_f
            ys = py[pl.ds(o, _LANES)] * r_f
            zs = pz[pl.ds(o, _LANES)] * r_f
            xi = jnp.minimum(xs.astype(jnp.int32), r - 1)
            yi = jnp.minimum(ys.astype(jnp.int32), r - 1)
            zi = jnp.minimum(zs.astype(jnp.int32), r - 1)
            fx = xs - xi.astype(jnp.float32)
            fy = ys - yi.astype(jnp.float32)
            fz = zs - zi.astype(jnp.float32)
            if hashed:
                hx = (xi, xi + 1)
                hy0 = yi * _P1
                hz0 = zi * _P2
                hy = (hy0, hy0 + _P1)
                hz = (hz0, hz0 + _P2)
                idxs = [((hx[c & 1] ^ hy[(c >> 1) & 1] ^ hz[(c >> 2) & 1])
                         & _MASK) + off
                        for c in range(8)]
            else:
                xc = (xi, xi + 1)
                ys0 = yi * s1
                zs0 = zi * s2
                yc = (ys0, ys0 + s1)
                zc = (zs0, zs0 + s2)
                idxs = [xc[c & 1] + yc[(c >> 1) & 1] + zc[(c >> 2) & 1] + off
                        for c in range(8)]
            for c in range(8):
                ib2[s, c, pl.ds(o, _LANES)] = lax.shift_right_logical(
                    idxs[c], 3)
                im2[s, c, pl.ds(o, _LANES)] = idxs[c] & 7
            wx = (1.0 - fx, fx)
            wy = (1.0 - fy, fy)
            wz = (1.0 - fz, fz)
            wxy = [wx[b0] * wy[b1] for b1 in range(2) for b0 in range(2)]
            for c in range(8):
                wb2[s, c, pl.ds(o, _LANES)] = (
                    wxy[((c >> 1) & 1) * 2 + (c & 1)] * wz[(c >> 2) & 1])
            return c2

        lax.fori_loop(0, _C // _LANES, body, 0)

    def fire(s):
        return ([pltpu.async_copy(t0_h.at[ib2.at[s, c]], ra2.at[s, c],
                                  sems[s])
                 for c in range(8)]
                + [pltpu.async_copy(t1_h.at[ib2.at[s, c]], rb2.at[s, c],
                                    sems[s])
                   for c in range(8)])

    def acc(l, s):
        def body(i, c2):
            o = i * _LANES
            ipt = iota + jnp.full((_LANES,), o, jnp.int32)
            acc0 = None
            acc1 = None
            for c in range(8):
                mv = im2[s, c, pl.ds(o, _LANES)]
                wv = wb2[s, c, pl.ds(o, _LANES)]
                r0 = plsc.load_gather(ra2.at[s, c], [ipt, mv])
                r1 = plsc.load_gather(rb2.at[s, c], [ipt, mv])
                t0 = wv * r0
                t1 = wv * r1
                acc0 = t0 if acc0 is None else acc0 + t0
                acc1 = t1 if acc1 is None else acc1 + t1
            encb[2 * l, pl.ds(o, _LANES)] = acc0
            encb[2 * l + 1, pl.ds(o, _LANES)] = acc1
            return c2

        lax.fori_loop(0, _C // _LANES, body, 0)

    def chunk_body(ch, carry):
        base = base0 + ch * _C
        pltpu.sync_copy(px_h.at[pl.ds(base, _C)], px)
        pltpu.sync_copy(py_h.at[pl.ds(base, _C)], py)
        pltpu.sync_copy(pz_h.at[pl.ds(base, _C)], pz)

        idxw(0, 0)
        descs = fire(0)
        for l in range(1, _NUM_LEVELS):
            s = l & 1
            idxw(l, s)
            nd = fire(s)
            for d in descs:
                d.wait()
            acc(l - 1, 1 - s)
            descs = nd
        for d in descs:
            d.wait()
        acc(_NUM_LEVELS - 1, (_NUM_LEVELS - 1) & 1)

        pltpu.sync_copy(encb, enc_h.at[:, pl.ds(base, _C)])
        return carry

    lax.fori_loop(0, _NCH, chunk_body, 0)


_enc_kernel = functools.partial(
    pl.kernel,
    out_type=jax.ShapeDtypeStruct((40, _N), jnp.float32),
    mesh=plsc.VectorSubcoreMesh(core_axis_name="c", subcore_axis_name="s",
                                num_cores=_NC, num_subcores=_NS),
    compiler_params=pltpu.CompilerParams(use_tc_tiling_on_sc=False,
                                         needs_layout_passes=False),
    scratch_types=(
        [pltpu.VMEM((_C,), jnp.float32)] * 3
        + [pltpu.VMEM((2, 8, _C), jnp.int32)] * 2
        + [pltpu.VMEM((2, 8, _C), jnp.float32)]
        + [pltpu.VMEM((2, 8, _C, 8), jnp.float32)] * 2
        + [pltpu.VMEM((40, _C), jnp.float32),
           pltpu.SemaphoreType.DMA, pltpu.SemaphoreType.DMA]
    ),
)(_enc_body)


_RB = 131072


def _split_body(t_ref, o0_ref, o1_ref):
    x = t_ref[...]
    o0_ref[...] = x[0, :]
    o1_ref[...] = x[1, :]


def _split_planes(table_t):
    grid = (_T + _RB - 1) // _RB
    return pl.pallas_call(
        _split_body,
        grid=(grid,),
        in_specs=[pl.BlockSpec((2, _RB), lambda i: (0, i))],
        out_specs=[pl.BlockSpec((_RB,), lambda i: (i,)),
                   pl.BlockSpec((_RB,), lambda i: (i,))],
        out_shape=[jax.ShapeDtypeStruct((_T,), jnp.float32),
                   jax.ShapeDtypeStruct((_T,), jnp.float32)],
    )(table_t)


_BT = 2048


def _mlp_body(e_ref, w1_ref, wo_ref, o_ref):
    x = jnp.maximum(
        jnp.dot(w1_ref[...], e_ref[...], preferred_element_type=jnp.float32),
        0.0)
    o_ref[...] = jnp.dot(wo_ref[...], x, preferred_element_type=jnp.float32)


def _mlp(enc, w1, wo):
    return pl.pallas_call(
        _mlp_body,
        grid=(_N // _BT,),
        in_specs=[
            pl.BlockSpec((40, _BT), lambda i: (0, i)),
            pl.BlockSpec((64, 40), lambda i: (0, 0)),
            pl.BlockSpec((8, 64), lambda i: (0, 0)),
        ],
        out_specs=pl.BlockSpec((8, _BT), lambda i: (0, i)),
        out_shape=jax.ShapeDtypeStruct((8, _N), jnp.float32),
    )(enc, w1, wo)


def kernel(positions, bound, table, W1, Wd, Wf):
    bf = jnp.float32(bound)
    pos01 = jnp.clip((positions + bf) / (2.0 * bf), 0.0, 1.0)
    px = pos01[:, 0]
    py = pos01[:, 1]
    pz = pos01[:, 2]
    t0, t1 = _split_planes(table.T)
    enc = _enc_kernel(px, py, pz,
                      t0.reshape(_T // 8, 8), t1.reshape(_T // 8, 8))
    wo = jnp.concatenate([Wf, Wd], axis=0)
    out8 = _mlp(enc, W1, wo)
    return (out8[:7, :].T, out8[7:8, :].T)


# speedup vs baseline: 3.5727x; 1.0003x over previous
import functools

import numpy as np
import jax
import jax.numpy as jnp
from jax import lax
from jax.experimental import pallas as pl
from jax.experimental.pallas import tpu as pltpu
from jax.experimental.pallas import tpu_sc as plsc

_NUM_LEVELS = 20
_LOG2_HASHMAP = 21
_BASE_RES = 16
_DESIRED_RES = 8193
_N = 262144
_P1 = np.uint32(2654435761).astype(np.int32)
_P2 = np.int32(805459861)
_MASK = np.int32(2**_LOG2_HASHMAP - 1)
_NDENSE = 7

_LANES = 16
_NC = 2
_NS = 16
_NW = _NC * _NS
_PW = _N // _NW
_C = 256
_NCH = _PW // _C


def _levels_meta():
    scale = np.exp(np.log(_DESIRED_RES / _BASE_RES) / (_NUM_LEVELS - 1))
    maxp = 2 ** _LOG2_HASHMAP
    res, sizes = [], []
    for i in range(_NUM_LEVELS):
        r = int(np.ceil(_BASE_RES * scale ** i))
        res.append(r)
        n = min(maxp, (r + 1) ** 3)
        sizes.append(int(np.ceil(n / 8.0) * 8))
    offs = [0]
    for s in sizes:
        offs.append(offs[-1] + s)
    return res, offs


_RES, _OFFS = _levels_meta()
_T = _OFFS[-1]
_TPAD = (-_T) % 128
_TP = _T + _TPAD


def _enc_body(px_h, py_h, pz_h, t0_h, t1_h, enc_h,
              px, py, pz, ib2, im2, wb2, ra2, rb2, encb, sem0, sem1):
    sems = (sem0, sem1)
    wid = lax.axis_index("s") * _NC + lax.axis_index("c")
    iota = lax.iota(jnp.int32, _LANES)
    base0 = wid * _PW

    def idxw(l, s):
        r = _RES[l]
        off = _OFFS[l]
        hashed = l >= _NDENSE
        r_f = float(r)
        s1 = r + 1
        s2 = s1 * s1

        def body(i, c2):
            o = i * _LANES
            xs = px[pl.ds(o, _LANES)] * r_f
            ys = py[pl.ds(o, _LANES)] * r_f
            zs = pz[pl.ds(o, _LANES)] * r_f
            xi = jnp.minimum(xs.astype(jnp.int32), r - 1)
            yi = jnp.minimum(ys.astype(jnp.int32), r - 1)
            zi = jnp.minimum(zs.astype(jnp.int32), r - 1)
            fx = xs - xi.astype(jnp.float32)
            fy = ys - yi.astype(jnp.float32)
            fz = zs - zi.astype(jnp.float32)
            if hashed:
                hx = (xi, xi + 1)
                hy0 = yi * _P1
                hz0 = zi * _P2
                hy = (hy0, hy0 + _P1)
                hz = (hz0, hz0 + _P2)
                idxs = [((hx[c & 1] ^ hy[(c >> 1) & 1] ^ hz[(c >> 2) & 1])
                         & _MASK) + off
                        for c in range(8)]
            else:
                xc = (xi, xi + 1)
                ys0 = yi * s1
                zs0 = zi * s2
                yc = (ys0, ys0 + s1)
                zc = (zs0, zs0 + s2)
                idxs = [xc[c & 1] + yc[(c >> 1) & 1] + zc[(c >> 2) & 1] + off
                        for c in range(8)]
            for c in range(8):
                ib2[s, pl.ds(c * _C + o, _LANES)] = lax.shift_right_logical(
                    idxs[c], 3)
                im2[s, c, pl.ds(o, _LANES)] = idxs[c] & 7
            wx = (1.0 - fx, fx)
            wy = (1.0 - fy, fy)
            wz = (1.0 - fz, fz)
            wxy = [wx[b0] * wy[b1] for b1 in range(2) for b0 in range(2)]
            for c in range(8):
                wb2[s, c, pl.ds(o, _LANES)] = (
                    wxy[((c >> 1) & 1) * 2 + (c & 1)] * wz[(c >> 2) & 1])
            return c2

        lax.fori_loop(0, _C // _LANES, body, 0)

    def fire(s):
        return [pltpu.async_copy(t0_h.at[ib2.at[s]], ra2.at[s], sems[s]),
                pltpu.async_copy(t1_h.at[ib2.at[s]], rb2.at[s], sems[s])]

    def acc(l, s):
        def body(i, c2):
            o = i * _LANES
            ipt = iota + jnp.full((_LANES,), o, jnp.int32)
            acc0 = None
            acc1 = None
            for c in range(8):
                mv = im2[s, c, pl.ds(o, _LANES)]
                wv = wb2[s, c, pl.ds(o, _LANES)]
                iptc = ipt + jnp.full((_LANES,), c * _C, jnp.int32)
                r0 = plsc.load_gather(ra2.at[s], [iptc, mv])
                r1 = plsc.load_gather(rb2.at[s], [iptc, mv])
                t0 = wv * r0
                t1 = wv * r1
                acc0 = t0 if acc0 is None else acc0 + t0
                acc1 = t1 if acc1 is None else acc1 + t1
            encb[2 * l, pl.ds(o, _LANES)] = acc0
            encb[2 * l + 1, pl.ds(o, _LANES)] = acc1
            return c2

        lax.fori_loop(0, _C // _LANES, body, 0)

    def chunk_body(ch, carry):
        base = base0 + ch * _C
        pltpu.sync_copy(px_h.at[pl.ds(base, _C)], px)
        pltpu.sync_copy(py_h.at[pl.ds(base, _C)], py)
        pltpu.sync_copy(pz_h.at[pl.ds(base, _C)], pz)

        idxw(0, 0)
        descs = fire(0)
        for l in range(1, _NUM_LEVELS):
            s = l & 1
            idxw(l, s)
            nd = fire(s)
            for d in descs:
                d.wait()
            acc(l - 1, 1 - s)
            descs = nd
        for d in descs:
            d.wait()
        acc(_NUM_LEVELS - 1, (_NUM_LEVELS - 1) & 1)

        pltpu.sync_copy(encb, enc_h.at[:, pl.ds(base, _C)])
        return carry

    lax.fori_loop(0, _NCH, chunk_body, 0)


_enc_kernel = functools.partial(
    pl.kernel,
    out_type=jax.ShapeDtypeStruct((40, _N), jnp.float32),
    mesh=plsc.VectorSubcoreMesh(core_axis_name="c", subcore_axis_name="s",
                                num_cores=_NC, num_subcores=_NS),
    compiler_params=pltpu.CompilerParams(use_tc_tiling_on_sc=False,
                                         needs_layout_passes=False),
    scratch_types=(
        [pltpu.VMEM((_C,), jnp.float32)] * 3
        + [pltpu.VMEM((2, 8 * _C), jnp.int32),
           pltpu.VMEM((2, 8, _C), jnp.int32)]
        + [pltpu.VMEM((2, 8, _C), jnp.float32)]
        + [pltpu.VMEM((2, 8 * _C, 8), jnp.float32)] * 2
        + [pltpu.VMEM((40, _C), jnp.float32),
           pltpu.SemaphoreType.DMA, pltpu.SemaphoreType.DMA]
    ),
)(_enc_body)


_RB = 131072


def _split_body(t_ref, o0_ref, o1_ref):
    x = t_ref[...]
    o0_ref[...] = x[0, :]
    o1_ref[...] = x[1, :]


def _split_planes(table_t):
    grid = (_T + _RB - 1) // _RB
    return pl.pallas_call(
        _split_body,
        grid=(grid,),
        in_specs=[pl.BlockSpec((2, _RB), lambda i: (0, i))],
        out_specs=[pl.BlockSpec((_RB,), lambda i: (i,)),
                   pl.BlockSpec((_RB,), lambda i: (i,))],
        out_shape=[jax.ShapeDtypeStruct((_T,), jnp.float32),
                   jax.ShapeDtypeStruct((_T,), jnp.float32)],
    )(table_t)


_BT = 2048


def _mlp_body(e_ref, w1_ref, wo_ref, o_ref):
    x = jnp.maximum(
        jnp.dot(w1_ref[...], e_ref[...], preferred_element_type=jnp.float32),
        0.0)
    o_ref[...] = jnp.dot(wo_ref[...], x, preferred_element_type=jnp.float32)


def _mlp(enc, w1, wo):
    return pl.pallas_call(
        _mlp_body,
        grid=(_N // _BT,),
        in_specs=[
            pl.BlockSpec((40, _BT), lambda i: (0, i)),
            pl.BlockSpec((64, 40), lambda i: (0, 0)),
            pl.BlockSpec((8, 64), lambda i: (0, 0)),
        ],
        out_specs=pl.BlockSpec((8, _BT), lambda i: (0, i)),
        out_shape=jax.ShapeDtypeStruct((8, _N), jnp.float32),
    )(enc, w1, wo)


def kernel(positions, bound, table, W1, Wd, Wf):
    bf = jnp.float32(bound)
    pos01 = jnp.clip((positions + bf) / (2.0 * bf), 0.0, 1.0)
    px = pos01[:, 0]
    py = pos01[:, 1]
    pz = pos01[:, 2]
    t0, t1 = _split_planes(table.T)
    enc = _enc_kernel(px, py, pz,
                      t0.reshape(_T // 8, 8), t1.reshape(_T // 8, 8))
    wo = jnp.concatenate([Wf, Wd], axis=0)
    out8 = _mlp(enc, W1, wo)
    return (out8[:7, :].T, out8[7:8, :].T)


# speedup vs baseline: 4.7888x; 1.3404x over previous
import functools

import numpy as np
import jax
import jax.numpy as jnp
from jax import lax
from jax.experimental import pallas as pl
from jax.experimental.pallas import tpu as pltpu
from jax.experimental.pallas import tpu_sc as plsc

_NUM_LEVELS = 20
_LOG2_HASHMAP = 21
_BASE_RES = 16
_DESIRED_RES = 8193
_N = 262144
_P1 = np.uint32(2654435761).astype(np.int32)
_P2 = np.int32(805459861)
_MASK = np.int32(2**_LOG2_HASHMAP - 1)
_NDENSE = 7

_LANES = 16
_NC = 2
_NS = 16
_NW = _NC * _NS
_PW = _N // _NW
_C = 256
_NCH = _PW // _C


def _levels_meta():
    scale = np.exp(np.log(_DESIRED_RES / _BASE_RES) / (_NUM_LEVELS - 1))
    maxp = 2 ** _LOG2_HASHMAP
    res, sizes = [], []
    for i in range(_NUM_LEVELS):
        r = int(np.ceil(_BASE_RES * scale ** i))
        res.append(r)
        n = min(maxp, (r + 1) ** 3)
        sizes.append(int(np.ceil(n / 8.0) * 8))
    offs = [0]
    for s in sizes:
        offs.append(offs[-1] + s)
    return res, offs


_RES, _OFFS = _levels_meta()
_T = _OFFS[-1]
_TPAD = (-_T) % 128
_TP = _T + _TPAD


def _enc_body(px_h, py_h, pz_h, tab_h, enc_h,
              px, py, pz, ib2, im2, wb2, ra2, encb, sem0, sem1):
    sems = (sem0, sem1)
    wid = lax.axis_index("s") * _NC + lax.axis_index("c")
    iota = lax.iota(jnp.int32, _LANES)
    base0 = wid * _PW

    def idxw(l, s):
        r = _RES[l]
        off = _OFFS[l]
        hashed = l >= _NDENSE
        r_f = float(r)
        s1 = r + 1
        s2 = s1 * s1

        def body(i, c2):
            o = i * _LANES
            xs = px[pl.ds(o, _LANES)] * r_f
            ys = py[pl.ds(o, _LANES)] * r_f
            zs = pz[pl.ds(o, _LANES)] * r_f
            xi = jnp.minimum(xs.astype(jnp.int32), r - 1)
            yi = jnp.minimum(ys.astype(jnp.int32), r - 1)
            zi = jnp.minimum(zs.astype(jnp.int32), r - 1)
            fx = xs - xi.astype(jnp.float32)
            fy = ys - yi.astype(jnp.float32)
            fz = zs - zi.astype(jnp.float32)
            if hashed:
                hx = (xi, xi + 1)
                hy0 = yi * _P1
                hz0 = zi * _P2
                hy = (hy0, hy0 + _P1)
                hz = (hz0, hz0 + _P2)
                idxs = [((hx[c & 1] ^ hy[(c >> 1) & 1] ^ hz[(c >> 2) & 1])
                         & _MASK) + off
                        for c in range(8)]
            else:
                xc = (xi, xi + 1)
                ys0 = yi * s1
                zs0 = zi * s2
                yc = (ys0, ys0 + s1)
                zc = (zs0, zs0 + s2)
                idxs = [xc[c & 1] + yc[(c >> 1) & 1] + zc[(c >> 2) & 1] + off
                        for c in range(8)]
            for c in range(8):
                ib2[s, pl.ds(c * _C + o, _LANES)] = lax.shift_right_logical(
                    idxs[c], 2)
                im2[s, c, pl.ds(o, _LANES)] = lax.shift_left(idxs[c] & 3, 1)
            wx = (1.0 - fx, fx)
            wy = (1.0 - fy, fy)
            wz = (1.0 - fz, fz)
            wxy = [wx[b0] * wy[b1] for b1 in range(2) for b0 in range(2)]
            for c in range(8):
                wb2[s, c, pl.ds(o, _LANES)] = (
                    wxy[((c >> 1) & 1) * 2 + (c & 1)] * wz[(c >> 2) & 1])
            return c2

        lax.fori_loop(0, _C // _LANES, body, 0)

    def fire(s):
        return [pltpu.async_copy(tab_h.at[ib2.at[s]], ra2.at[s], sems[s])]

    def acc(l, s):
        def body(i, c2):
            o = i * _LANES
            ipt = iota + jnp.full((_LANES,), o, jnp.int32)
            acc0 = None
            acc1 = None
            for c in range(8):
                mv = im2[s, c, pl.ds(o, _LANES)]
                wv = wb2[s, c, pl.ds(o, _LANES)]
                iptc = ipt + jnp.full((_LANES,), c * _C, jnp.int32)
                r0 = plsc.load_gather(ra2.at[s], [iptc, mv])
                r1 = plsc.load_gather(ra2.at[s], [iptc, mv + 1])
                t0 = wv * r0
                t1 = wv * r1
                acc0 = t0 if acc0 is None else acc0 + t0
                acc1 = t1 if acc1 is None else acc1 + t1
            encb[2 * l, pl.ds(o, _LANES)] = acc0
            encb[2 * l + 1, pl.ds(o, _LANES)] = acc1
            return c2

        lax.fori_loop(0, _C // _LANES, body, 0)

    def chunk_body(ch, carry):
        base = base0 + ch * _C
        pltpu.sync_copy(px_h.at[pl.ds(base, _C)], px)
        pltpu.sync_copy(py_h.at[pl.ds(base, _C)], py)
        pltpu.sync_copy(pz_h.at[pl.ds(base, _C)], pz)

        idxw(0, 0)
        descs = fire(0)
        for l in range(1, _NUM_LEVELS):
            s = l & 1
            idxw(l, s)
            nd = fire(s)
            for d in descs:
                d.wait()
            acc(l - 1, 1 - s)
            descs = nd
        for d in descs:
            d.wait()
        acc(_NUM_LEVELS - 1, (_NUM_LEVELS - 1) & 1)

        pltpu.sync_copy(encb, enc_h.at[:, pl.ds(base, _C)])
        return carry

    lax.fori_loop(0, _NCH, chunk_body, 0)


_enc_kernel = functools.partial(
    pl.kernel,
    out_type=jax.ShapeDtypeStruct((40, _N), jnp.float32),
    mesh=plsc.VectorSubcoreMesh(core_axis_name="c", subcore_axis_name="s",
                                num_cores=_NC, num_subcores=_NS),
    compiler_params=pltpu.CompilerParams(use_tc_tiling_on_sc=False,
                                         needs_layout_passes=False),
    scratch_types=(
        [pltpu.VMEM((_C,), jnp.float32)] * 3
        + [pltpu.VMEM((2, 8 * _C), jnp.int32),
           pltpu.VMEM((2, 8, _C), jnp.int32)]
        + [pltpu.VMEM((2, 8, _C), jnp.float32)]
        + [pltpu.VMEM((2, 8 * _C, 8), jnp.float32)]
        + [pltpu.VMEM((40, _C), jnp.float32),
           pltpu.SemaphoreType.DMA, pltpu.SemaphoreType.DMA]
    ),
)(_enc_body)


_RB = 131072


def _split_body(t_ref, o0_ref, o1_ref):
    x = t_ref[...]
    o0_ref[...] = x[0, :]
    o1_ref[...] = x[1, :]


_GRID = (_T + _RB - 1) // _RB
_TR = _GRID * _RB


def _split_planes(table_t):
    return pl.pallas_call(
        _split_body,
        grid=(_GRID,),
        in_specs=[pl.BlockSpec((2, _RB), lambda i: (0, i))],
        out_specs=[pl.BlockSpec((_RB,), lambda i: (i,)),
                   pl.BlockSpec((_RB,), lambda i: (i,))],
        out_shape=[jax.ShapeDtypeStruct((_TR,), jnp.float32),
                   jax.ShapeDtypeStruct((_TR,), jnp.float32)],
    )(table_t)


_K = 16384
_NRCH = _TR // _K // _NW


def _rp_body(t0_h, t1_h, out_h, v0, v1, vout):
    wid = lax.axis_index("s") * _NC + lax.axis_index("c")
    iota = lax.iota(jnp.int32, _LANES)
    iota2 = iota + iota

    def body(j, carry):
        g = wid * _NRCH + j
        s0 = g * _K
        pltpu.sync_copy(t0_h.at[pl.ds(s0, _K)], v0)
        pltpu.sync_copy(t1_h.at[pl.ds(s0, _K)], v1)

        def il(i, c2):
            o = i * _LANES
            ip = iota2 + jnp.full((_LANES,), 2 * o, jnp.int32)
            plsc.store_scatter(vout, [ip], v0[pl.ds(o, _LANES)])
            plsc.store_scatter(vout, [ip + 1], v1[pl.ds(o, _LANES)])
            return c2

        lax.fori_loop(0, _K // _LANES, il, 0)
        pltpu.sync_copy(vout, out_h.at[pl.ds(2 * _K * g, 2 * _K)])
        return carry

    lax.fori_loop(0, _NRCH, body, 0)


_repack = functools.partial(
    pl.kernel,
    out_type=jax.ShapeDtypeStruct((2 * _TR,), jnp.float32),
    mesh=plsc.VectorSubcoreMesh(core_axis_name="c", subcore_axis_name="s",
                                num_cores=_NC, num_subcores=_NS),
    compiler_params=pltpu.CompilerParams(use_tc_tiling_on_sc=False,
                                         needs_layout_passes=False),
    scratch_types=[
        pltpu.VMEM((_K,), jnp.float32),
        pltpu.VMEM((_K,), jnp.float32),
        pltpu.VMEM((2 * _K,), jnp.float32),
    ],
)(_rp_body)


_BT = 2048


def _mlp_body(e_ref, w1_ref, wo_ref, o_ref):
    x = jnp.maximum(
        jnp.dot(w1_ref[...], e_ref[...], preferred_element_type=jnp.float32),
        0.0)
    o_ref[...] = jnp.dot(wo_ref[...], x, preferred_element_type=jnp.float32)


def _mlp(enc, w1, wo):
    return pl.pallas_call(
        _mlp_body,
        grid=(_N // _BT,),
        in_specs=[
            pl.BlockSpec((40, _BT), lambda i: (0, i)),
            pl.BlockSpec((64, 40), lambda i: (0, 0)),
            pl.BlockSpec((8, 64), lambda i: (0, 0)),
        ],
        out_specs=pl.BlockSpec((8, _BT), lambda i: (0, i)),
        out_shape=jax.ShapeDtypeStruct((8, _N), jnp.float32),
    )(enc, w1, wo)


def kernel(positions, bound, table, W1, Wd, Wf):
    bf = jnp.float32(bound)
    pos01 = jnp.clip((positions + bf) / (2.0 * bf), 0.0, 1.0)
    px = pos01[:, 0]
    py = pos01[:, 1]
    pz = pos01[:, 2]
    t0, t1 = _split_planes(table.T)
    tcat = _repack(t0, t1).reshape(_TR // 4, 8)
    enc = _enc_kernel(px, py, pz, tcat)
    wo = jnp.concatenate([Wf, Wd], axis=0)
    out8 = _mlp(enc, W1, wo)
    return (out8[:7, :].T, out8[7:8, :].T)


# speedup vs baseline: 5.5216x; 1.1530x over previous
import functools

import numpy as np
import jax
import jax.numpy as jnp
from jax import lax
from jax.experimental import pallas as pl
from jax.experimental.pallas import tpu as pltpu
from jax.experimental.pallas import tpu_sc as plsc

_NUM_LEVELS = 20
_LOG2_HASHMAP = 21
_BASE_RES = 16
_DESIRED_RES = 8193
_N = 262144
_P1 = np.uint32(2654435761).astype(np.int32)
_P2 = np.int32(805459861)
_MASK = np.int32(2**_LOG2_HASHMAP - 1)
_NDENSE = 7

_LANES = 16
_NC = 2
_NS = 16
_NW = _NC * _NS
_PW = _N // _NW
_C = 256
_NCH = _PW // _C


def _levels_meta():
    scale = np.exp(np.log(_DESIRED_RES / _BASE_RES) / (_NUM_LEVELS - 1))
    maxp = 2 ** _LOG2_HASHMAP
    res, sizes = [], []
    for i in range(_NUM_LEVELS):
        r = int(np.ceil(_BASE_RES * scale ** i))
        res.append(r)
        n = min(maxp, (r + 1) ** 3)
        sizes.append(int(np.ceil(n / 8.0) * 8))
    offs = [0]
    for s in sizes:
        offs.append(offs[-1] + s)
    return res, offs


_RES, _OFFS = _levels_meta()
_T = _OFFS[-1]
_NCACHED = 2
_CBLK = (_OFFS[_NCACHED] * 2) // 8
_TPAD = (-_T) % 128
_TP = _T + _TPAD


def _enc_body(px_h, py_h, pz_h, tab_h, enc_h,
              px, py, pz, ib2, im2, wb2, ra2, tcache, encb, sem0, sem1):
    sems = (sem0, sem1)
    wid = lax.axis_index("s") * _NC + lax.axis_index("c")
    iota = lax.iota(jnp.int32, _LANES)
    base0 = wid * _PW
    pltpu.sync_copy(tab_h.at[pl.ds(0, _CBLK), :], tcache)

    def idxw(l, s):
        r = _RES[l]
        off = _OFFS[l]
        hashed = l >= _NDENSE
        r_f = float(r)
        s1 = r + 1
        s2 = s1 * s1

        def body(i, c2):
            o = i * _LANES
            xs = px[pl.ds(o, _LANES)] * r_f
            ys = py[pl.ds(o, _LANES)] * r_f
            zs = pz[pl.ds(o, _LANES)] * r_f
            xi = jnp.minimum(xs.astype(jnp.int32), r - 1)
            yi = jnp.minimum(ys.astype(jnp.int32), r - 1)
            zi = jnp.minimum(zs.astype(jnp.int32), r - 1)
            fx = xs - xi.astype(jnp.float32)
            fy = ys - yi.astype(jnp.float32)
            fz = zs - zi.astype(jnp.float32)
            if hashed:
                hx = (xi, xi + 1)
                hy0 = yi * _P1
                hz0 = zi * _P2
                hy = (hy0, hy0 + _P1)
                hz = (hz0, hz0 + _P2)
                idxs = [((hx[c & 1] ^ hy[(c >> 1) & 1] ^ hz[(c >> 2) & 1])
                         & _MASK) + off
                        for c in range(8)]
            else:
                xc = (xi, xi + 1)
                ys0 = yi * s1
                zs0 = zi * s2
                yc = (ys0, ys0 + s1)
                zc = (zs0, zs0 + s2)
                idxs = [xc[c & 1] + yc[(c >> 1) & 1] + zc[(c >> 2) & 1] + off
                        for c in range(8)]
            for c in range(8):
                ib2[s, pl.ds(c * _C + o, _LANES)] = lax.shift_right_logical(
                    idxs[c], 2)
                im2[s, c, pl.ds(o, _LANES)] = lax.shift_left(idxs[c] & 3, 1)
            wx = (1.0 - fx, fx)
            wy = (1.0 - fy, fy)
            wz = (1.0 - fz, fz)
            wxy = [wx[b0] * wy[b1] for b1 in range(2) for b0 in range(2)]
            for c in range(8):
                wb2[s, c, pl.ds(o, _LANES)] = (
                    wxy[((c >> 1) & 1) * 2 + (c & 1)] * wz[(c >> 2) & 1])
            return c2

        lax.fori_loop(0, _C // _LANES, body, 0)

    def fire(s):
        return [pltpu.async_copy(tab_h.at[ib2.at[s]], ra2.at[s], sems[s])]

    def acc(l, s):
        cached = l < _NCACHED

        def body(i, c2):
            o = i * _LANES
            ipt = iota + jnp.full((_LANES,), o, jnp.int32)
            acc0 = None
            acc1 = None
            for c in range(8):
                mv = im2[s, c, pl.ds(o, _LANES)]
                wv = wb2[s, c, pl.ds(o, _LANES)]
                if cached:
                    bv = ib2[s, pl.ds(c * _C + o, _LANES)]
                    r0 = plsc.load_gather(tcache, [bv, mv])
                    r1 = plsc.load_gather(tcache, [bv, mv + 1])
                else:
                    iptc = ipt + jnp.full((_LANES,), c * _C, jnp.int32)
                    r0 = plsc.load_gather(ra2.at[s], [iptc, mv])
                    r1 = plsc.load_gather(ra2.at[s], [iptc, mv + 1])
                t0 = wv * r0
                t1 = wv * r1
                acc0 = t0 if acc0 is None else acc0 + t0
                acc1 = t1 if acc1 is None else acc1 + t1
            encb[2 * l, pl.ds(o, _LANES)] = acc0
            encb[2 * l + 1, pl.ds(o, _LANES)] = acc1
            return c2

        lax.fori_loop(0, _C // _LANES, body, 0)

    def chunk_body(ch, carry):
        base = base0 + ch * _C
        pltpu.sync_copy(px_h.at[pl.ds(base, _C)], px)
        pltpu.sync_copy(py_h.at[pl.ds(base, _C)], py)
        pltpu.sync_copy(pz_h.at[pl.ds(base, _C)], pz)

        idxw(0, 0)
        descs = []
        for l in range(1, _NUM_LEVELS):
            s = l & 1
            idxw(l, s)
            nd = fire(s) if l >= _NCACHED else []
            for d in descs:
                d.wait()
            acc(l - 1, 1 - s)
            descs = nd
        for d in descs:
            d.wait()
        acc(_NUM_LEVELS - 1, (_NUM_LEVELS - 1) & 1)

        pltpu.sync_copy(encb, enc_h.at[:, pl.ds(base, _C)])
        return carry

    lax.fori_loop(0, _NCH, chunk_body, 0)


_enc_kernel = functools.partial(
    pl.kernel,
    out_type=jax.ShapeDtypeStruct((40, _N), jnp.float32),
    mesh=plsc.VectorSubcoreMesh(core_axis_name="c", subcore_axis_name="s",
                                num_cores=_NC, num_subcores=_NS),
    compiler_params=pltpu.CompilerParams(use_tc_tiling_on_sc=False,
                                         needs_layout_passes=False),
    scratch_types=(
        [pltpu.VMEM((_C,), jnp.float32)] * 3
        + [pltpu.VMEM((2, 8 * _C), jnp.int32),
           pltpu.VMEM((2, 8, _C), jnp.int32)]
        + [pltpu.VMEM((2, 8, _C), jnp.float32)]
        + [pltpu.VMEM((2, 8 * _C, 8), jnp.float32)]
        + [pltpu.VMEM((_CBLK, 8), jnp.float32)]
        + [pltpu.VMEM((40, _C), jnp.float32),
           pltpu.SemaphoreType.DMA, pltpu.SemaphoreType.DMA]
    ),
)(_enc_body)


_RB = 131072


def _split_body(t_ref, o0_ref, o1_ref):
    x = t_ref[...]
    o0_ref[...] = x[0, :]
    o1_ref[...] = x[1, :]


_GRID = (_T + _RB - 1) // _RB
_TR = _GRID * _RB


def _split_planes(table_t):
    return pl.pallas_call(
        _split_body,
        grid=(_GRID,),
        in_specs=[pl.BlockSpec((2, _RB), lambda i: (0, i))],
        out_specs=[pl.BlockSpec((_RB,), lambda i: (i,)),
                   pl.BlockSpec((_RB,), lambda i: (i,))],
        out_shape=[jax.ShapeDtypeStruct((_TR,), jnp.float32),
                   jax.ShapeDtypeStruct((_TR,), jnp.float32)],
    )(table_t)


_K = 8192
_NRCH = _TR // _K // _NW
_NRTOT = _TR // _K


def _rp_body(t0_h, t1_h, out_h, v0, v1, vout, semi0, semi1):
    wid = lax.axis_index("s") * _NC + lax.axis_index("c")
    iota = lax.iota(jnp.int32, _LANES)
    iota2 = iota + iota
    semis = (semi0, semi1)
    g0 = wid * _NRCH

    def start_in(g, s):
        gc = jnp.minimum(g, _NRTOT - 1)
        return [pltpu.async_copy(t0_h.at[pl.ds(gc * _K, _K)], v0.at[s],
                                 semis[s]),
                pltpu.async_copy(t1_h.at[pl.ds(gc * _K, _K)], v1.at[s],
                                 semis[s])]

    def work(g, s):
        def il(i, c2):
            o = i * _LANES
            ip = iota2 + jnp.full((_LANES,), 2 * o, jnp.int32)
            plsc.store_scatter(vout, [ip], v0[s, pl.ds(o, _LANES)])
            plsc.store_scatter(vout, [ip + 1], v1[s, pl.ds(o, _LANES)])
            return c2

        lax.fori_loop(0, _K // _LANES, il, 0)
        pltpu.sync_copy(vout, out_h.at[pl.ds(2 * _K * g, 2 * _K)])

    descs0 = start_in(g0, 0)

    def body(j2, carry):
        j = j2 * 2
        for d in descs0:
            d.wait()
        d1 = start_in(g0 + j + 1, 1)
        work(g0 + j, 0)
        for d in d1:
            d.wait()
        start_in(g0 + j + 2, 0)
        work(g0 + j + 1, 1)
        return carry

    lax.fori_loop(0, _NRCH // 2, body, 0)
    pltpu.make_async_copy(t0_h.at[pl.ds(0, _K)], v0.at[0], semi0).wait()
    pltpu.make_async_copy(t1_h.at[pl.ds(0, _K)], v1.at[0], semi0).wait()


_repack = functools.partial(
    pl.kernel,
    out_type=jax.ShapeDtypeStruct((2 * _TR,), jnp.float32),
    mesh=plsc.VectorSubcoreMesh(core_axis_name="c", subcore_axis_name="s",
                                num_cores=_NC, num_subcores=_NS),
    compiler_params=pltpu.CompilerParams(use_tc_tiling_on_sc=False,
                                         needs_layout_passes=False),
    scratch_types=[
        pltpu.VMEM((2, _K), jnp.float32),
        pltpu.VMEM((2, _K), jnp.float32),
        pltpu.VMEM((2 * _K,), jnp.float32),
        pltpu.SemaphoreType.DMA, pltpu.SemaphoreType.DMA,
    ],
)(_rp_body)


_BT = 2048


def _mlp_body(e_ref, w1_ref, wo_ref, o_ref):
    x = jnp.maximum(
        jnp.dot(w1_ref[...], e_ref[...], preferred_element_type=jnp.float32),
        0.0)
    o_ref[...] = jnp.dot(wo_ref[...], x, preferred_element_type=jnp.float32)


def _mlp(enc, w1, wo):
    return pl.pallas_call(
        _mlp_body,
        grid=(_N // _BT,),
        in_specs=[
            pl.BlockSpec((40, _BT), lambda i: (0, i)),
            pl.BlockSpec((64, 40), lambda i: (0, 0)),
            pl.BlockSpec((8, 64), lambda i: (0, 0)),
        ],
        out_specs=pl.BlockSpec((8, _BT), lambda i: (0, i)),
        out_shape=jax.ShapeDtypeStruct((8, _N), jnp.float32),
    )(enc, w1, wo)


def kernel(positions, bound, table, W1, Wd, Wf):
    bf = jnp.float32(bound)
    pos01 = jnp.clip((positions + bf) / (2.0 * bf), 0.0, 1.0)
    px = pos01[:, 0]
    py = pos01[:, 1]
    pz = pos01[:, 2]
    t0, t1 = _split_planes(table.T)
    tcat = _repack(t0, t1).reshape(_TR // 4, 8)
    enc = _enc_kernel(px, py, pz, tcat)
    wo = jnp.concatenate([Wf, Wd], axis=0)
    out8 = _mlp(enc, W1, wo)
    return (out8[:7, :].T, out8[7:8, :].T)
